# baseline (speedup 1.0000x reference)
import dataclasses
import functools

import jax
import jax.numpy as jnp
from jax import lax
from jax.experimental import pallas as pl
from jax.experimental.pallas import tpu as pltpu
from jax.experimental.pallas import tpu_sc as plsc

N = 10000
H = 256
HW = H // 2
S = 8
NC = 2
NS = 16
TPW0 = 640
TPW1 = 0
NPAD = NS * (TPW0 + TPW1)
BASE1 = NS * TPW0
C = 8
NBUF = 4
BLK = 1000
GRID = N // BLK

_PERM_LO = jnp.arange(HW) // 16 * 32 + jnp.arange(HW) % 16
_PERM_HI = _PERM_LO + 16


def _rne16(v):
    b = jax.lax.bitcast_convert_type(v, jnp.int32)
    tie = jax.lax.shift_right_logical(b, 16) & 1
    return jax.lax.shift_right_logical(b + 0x7FFF + tie, 16)


def _unpack_f32(x):
    lo = jax.lax.bitcast_convert_type(jax.lax.shift_left(x, 16), jnp.float32)
    hi = jax.lax.bitcast_convert_type(x & jnp.int32(-65536), jnp.float32)
    return lo, hi


def _tc_pack(h1, h2, ea, eb):
    def body(x0_ref, x1_ref, ea_ref, eb_ref, o0_ref, o1_ref):
        for x_ref, o_ref in ((x0_ref, o0_ref), (x1_ref, o1_ref)):
            x = x_ref[...]
            lo = jnp.dot(x, ea_ref[...], preferred_element_type=jnp.float32)
            hi = jnp.dot(x, eb_ref[...], preferred_element_type=jnp.float32)
            o_ref[...] = _rne16(lo) | jax.lax.shift_left(_rne16(hi), 16)

    return pl.pallas_call(
        body,
        grid=(GRID,),
        in_specs=[
            pl.BlockSpec((BLK, H), lambda i: (i, 0)),
            pl.BlockSpec((BLK, H), lambda i: (i, 0)),
            pl.BlockSpec((H, HW), lambda i: (0, 0)),
            pl.BlockSpec((H, HW), lambda i: (0, 0)),
        ],
        out_specs=[
            pl.BlockSpec((BLK, HW), lambda i: (i, 0)),
            pl.BlockSpec((BLK, HW), lambda i: (i, 0)),
        ],
        out_shape=[
            jax.ShapeDtypeStruct((N, HW), jnp.int32),
            jax.ShapeDtypeStruct((N, HW), jnp.int32),
        ],
    )(h1, h2, ea, eb)


def _sc_gather_sums(h1p, h2p, idx0, idx1):
    mesh = plsc.VectorSubcoreMesh(core_axis_name="c", subcore_axis_name="s")
    cp = pltpu.CompilerParams()
    if "needs_layout_passes" in pltpu.CompilerParams.__dataclass_fields__:
        cp = dataclasses.replace(cp, needs_layout_passes=False)

    @functools.partial(
        pl.kernel,
        compiler_params=cp,
        out_type=(
            jax.ShapeDtypeStruct((NPAD, HW), jnp.int32),
            jax.ShapeDtypeStruct((NPAD, HW), jnp.int32),
        ),
        mesh=mesh,
        scratch_types=(
            [pltpu.VMEM((2 * TPW0 * S,), jnp.int32)]
            + [pltpu.VMEM((C * S, HW), jnp.int32)] * NBUF
            + [pltpu.VMEM((C, HW), jnp.int32)] * NBUF
            + [pltpu.SemaphoreType.DMA] * (2 * NBUF + 2)
        ),
    )
    def sc_kernel(h1_hbm, h2_hbm, i0_hbm, i1_hbm, o0_hbm, o1_hbm,
                  idx_v, *bufs):
        rows = bufs[0:NBUF]
        accs = bufs[NBUF:2 * NBUF]
        gsem = bufs[2 * NBUF:3 * NBUF]
        wsem = bufs[3 * NBUF:4 * NBUF]
        isem = bufs[4 * NBUF:4 * NBUF + 2]
        core = lax.axis_index("c")
        sid = lax.axis_index("s")
        ioff = TPW0 * S

        for ci, tpw in ((0, TPW0), (1, TPW1)):
            if tpw == 0:
                continue
            chunks = tpw // C

            @pl.when(core == ci)
            def _(ci=ci, tpw=tpw, chunks=chunks):
                tbase = sid * tpw + (BASE1 if ci == 1 else 0)
                ibase = tbase * S

                def i_copy(i_hbm, tab, tab_isem):
                    return pltpu.make_async_copy(
                        i_hbm.at[pl.ds(ibase, tpw * S)],
                        idx_v.at[pl.ds(tab * ioff, tpw * S)], tab_isem)

                i_copy(i0_hbm, 0, isem[0]).start()
                i_copy(i1_hbm, 1, isem[1]).start()

                for tab, (t_hbm, i_hbm, o_hbm) in enumerate(
                        ((h1_hbm, i0_hbm, o0_hbm), (h2_hbm, i1_hbm, o1_hbm))):
                    i_copy(i_hbm, tab, isem[tab]).wait()

                    def g_start(cc, b, t_hbm=t_hbm, tab=tab):
                        pltpu.make_async_copy(
                            t_hbm.at[idx_v.at[
                                pl.ds(tab * ioff + cc * (C * S), C * S)]],
                            rows[b], gsem[b]).start()

                    def g_wait(b, t_hbm=t_hbm):
                        pltpu.make_async_copy(
                            t_hbm.at[idx_v.at[pl.ds(0, C * S)]],
                            rows[b], gsem[b]).wait()

                    def w_start(cc, b, o_hbm=o_hbm, tbase=tbase):
                        pltpu.make_async_copy(
                            accs[b], o_hbm.at[pl.ds(tbase + cc * C, C)],
                            wsem[b]).start()

                    def w_wait(b, o_hbm=o_hbm, tbase=tbase):
                        pltpu.make_async_copy(
                            accs[b], o_hbm.at[pl.ds(tbase, C)],
                            wsem[b]).wait()

                    for pre in range(NBUF - 1):
                        g_start(pre, pre)

                    @pl.loop(0, chunks, step=NBUF)
                    def _(c, g_start=g_start, g_wait=g_wait,
                          w_start=w_start, w_wait=w_wait, chunks=chunks):
                        for b in range(NBUF):
                            cc = c + b
                            nxt = cc + NBUF - 1

                            @pl.when(nxt < chunks)
                            def _(nxt=nxt, b=b):
                                g_start(nxt, (b + NBUF - 1) % NBUF)

                            g_wait(b)

                            @pl.when(cc >= NBUF)
                            def _(b=b):
                                w_wait(b)

                            rb, ab = rows[b], accs[b]

                            @pl.loop(0, C)
                            def _(t, rb=rb, ab=ab):
                                r = t * S

                                def bf(x):
                                    return plsc.bitcast(x, jnp.bfloat16)

                                for j in range(HW // 16):
                                    sl = pl.ds(j * 16, 16)
                                    v01 = bf(rb[r, sl]) + bf(rb[r + 1, sl])
                                    v23 = bf(rb[r + 2, sl]) + bf(rb[r + 3, sl])
                                    v45 = bf(rb[r + 4, sl]) + bf(rb[r + 5, sl])
                                    v67 = bf(rb[r + 6, sl]) + bf(rb[r + 7, sl])
                                    ab[t, sl] = plsc.bitcast(
                                        (v01 + v23) + (v45 + v67), jnp.int32)

                            w_start(cc, b)

                    for b in range(NBUF):
                        w_wait(b)

    return sc_kernel(h1p, h2p, idx0, idx1)


def _tc_colsums(s0, s1, wa, wb, fc_b):
    def body(x0_ref, x1_ref, wa_ref, wb_ref, b_ref, out_ref):
        @pl.when(pl.program_id(0) == 0)
        def _():
            out_ref[...] = jnp.zeros_like(out_ref)

        for i, x_ref in enumerate((x0_ref, x1_ref)):
            lo, hi = _unpack_f32(x_ref[...])
            t = jnp.tanh(
                jnp.dot(lo, wa_ref[...], preferred_element_type=jnp.float32)
                + jnp.dot(hi, wb_ref[...], preferred_element_type=jnp.float32)
                + b_ref[...])
            out_ref[i:i + 1, :] += jnp.sum(t, axis=0, keepdims=True)

    return pl.pallas_call(
        body,
        grid=(GRID,),
        in_specs=[
            pl.BlockSpec((BLK, HW), lambda i: (i, 0)),
            pl.BlockSpec((BLK, HW), lambda i: (i, 0)),
            pl.BlockSpec((HW, H), lambda i: (0, 0)),
            pl.BlockSpec((HW, H), lambda i: (0, 0)),
            pl.BlockSpec((1, H), lambda i: (0, 0)),
        ],
        out_specs=pl.BlockSpec((8, H), lambda i: (0, 0)),
        out_shape=jax.ShapeDtypeStruct((8, H), jnp.float32),
    )(s0, s1, wa, wb, fc_b)


def _tc_combine(cs, att, s0, s1, pa, pb, pred_b):
    def body(cs_ref, att_ref, x0_ref, x1_ref, pa_ref, pb_ref, b_ref, out_ref):
        a = att_ref[0, :]
        v0 = jnp.sum(cs_ref[0, :] * a) * (1.0 / N)
        v1 = jnp.sum(cs_ref[1, :] * a) * (1.0 / N)
        m = jnp.maximum(v0, v1)
        e0 = jnp.exp(v0 - m)
        e1 = jnp.exp(v1 - m)
        inv = 1.0 / (e0 + e1)
        b0 = e0 * inv
        b1 = e1 * inv
        lo0, hi0 = _unpack_f32(x0_ref[...])
        lo1, hi1 = _unpack_f32(x1_ref[...])
        z_lo = lo0 * b0 + lo1 * b1
        z_hi = hi0 * b0 + hi1 * b1
        out_ref[...] = jnp.tanh(
            jnp.dot(z_lo, pa_ref[...], preferred_element_type=jnp.float32)
            + jnp.dot(z_hi, pb_ref[...], preferred_element_type=jnp.float32)
            + b_ref[...])

    return pl.pallas_call(
        body,
        grid=(GRID,),
        in_specs=[
            pl.BlockSpec((8, H), lambda i: (0, 0)),
            pl.BlockSpec((1, H), lambda i: (0, 0)),
            pl.BlockSpec((BLK, HW), lambda i: (i, 0)),
            pl.BlockSpec((BLK, HW), lambda i: (i, 0)),
            pl.BlockSpec((HW, H), lambda i: (0, 0)),
            pl.BlockSpec((HW, H), lambda i: (0, 0)),
            pl.BlockSpec((1, H), lambda i: (0, 0)),
        ],
        out_specs=pl.BlockSpec((BLK, H), lambda i: (i, 0)),
        out_shape=jax.ShapeDtypeStruct((N, H), jnp.float32),
    )(cs, att, s0, s1, pa, pb, pred_b)


def kernel(h0, h1, h2, nei_idx0, nei_idx1, fc_W, fc_b, att, pred_W, pred_b):
    del h0
    idx0 = nei_idx0.astype(jnp.int32).reshape(-1)
    idx1 = nei_idx1.astype(jnp.int32).reshape(-1)
    pad = NPAD * S - idx0.shape[0]
    idx0 = jnp.concatenate([idx0, jnp.zeros((pad,), jnp.int32)])
    idx1 = jnp.concatenate([idx1, jnp.zeros((pad,), jnp.int32)])

    cols = jnp.arange(HW)
    ea = jnp.zeros((H, HW), jnp.float32).at[_PERM_LO, cols].set(1.0)
    eb = jnp.zeros((H, HW), jnp.float32).at[_PERM_HI, cols].set(1.0)

    h1p, h2p = _tc_pack(h1, h2, ea, eb)
    s0, s1 = _sc_gather_sums(h1p, h2p, idx0, idx1)

    fc_wt = fc_W.T * (1.0 / S)
    pred_wt = pred_W.T * (1.0 / S)
    cs = _tc_colsums(s0, s1, fc_wt[_PERM_LO, :], fc_wt[_PERM_HI, :],
                     fc_b.reshape(1, H))
    out = _tc_combine(cs, att.reshape(1, H), s0, s1,
                      pred_wt[_PERM_LO, :], pred_wt[_PERM_HI, :],
                      pred_b.reshape(1, H))
    return out


# speedup vs baseline: 1.5050x; 1.5050x over previous
import dataclasses
import functools

import jax
import jax.numpy as jnp
from jax import lax
from jax.experimental import pallas as pl
from jax.experimental.pallas import tpu as pltpu
from jax.experimental.pallas import tpu_sc as plsc

N = 10000
H = 256
HW = H // 2
S = 8
NC = 2
NS = 16
TPW0 = 608
TPW1 = 32
NPAD = NS * (TPW0 + TPW1)
BASE1 = NS * TPW0
C = 8
NBUF = 4
BLK = 1000
GRID = N // BLK

_PERM_LO = jnp.arange(HW) // 16 * 32 + jnp.arange(HW) % 16
_PERM_HI = _PERM_LO + 16


def _rne16(v):
    b = jax.lax.bitcast_convert_type(v, jnp.int32)
    tie = jax.lax.shift_right_logical(b, 16) & 1
    return jax.lax.shift_right_logical(b + 0x7FFF + tie, 16)


def _unpack_f32(x):
    lo = jax.lax.bitcast_convert_type(jax.lax.shift_left(x, 16), jnp.float32)
    hi = jax.lax.bitcast_convert_type(x & jnp.int32(-65536), jnp.float32)
    return lo, hi


def _tc_pack(h1, h2, ea, eb):
    def body(x0_ref, x1_ref, ea_ref, eb_ref, o0_ref, o1_ref):
        for x_ref, o_ref in ((x0_ref, o0_ref), (x1_ref, o1_ref)):
            x = x_ref[...]
            lo = jnp.dot(x, ea_ref[...], preferred_element_type=jnp.float32)
            hi = jnp.dot(x, eb_ref[...], preferred_element_type=jnp.float32)
            o_ref[...] = _rne16(lo) | jax.lax.shift_left(_rne16(hi), 16)

    return pl.pallas_call(
        body,
        grid=(GRID,),
        in_specs=[
            pl.BlockSpec((BLK, H), lambda i: (i, 0)),
            pl.BlockSpec((BLK, H), lambda i: (i, 0)),
            pl.BlockSpec((H, HW), lambda i: (0, 0)),
            pl.BlockSpec((H, HW), lambda i: (0, 0)),
        ],
        out_specs=[
            pl.BlockSpec((BLK, HW), lambda i: (i, 0)),
            pl.BlockSpec((BLK, HW), lambda i: (i, 0)),
        ],
        out_shape=[
            jax.ShapeDtypeStruct((N, HW), jnp.int32),
            jax.ShapeDtypeStruct((N, HW), jnp.int32),
        ],
    )(h1, h2, ea, eb)


def _sc_gather_sums(h1p, h2p, idx0, idx1):
    mesh = plsc.VectorSubcoreMesh(core_axis_name="c", subcore_axis_name="s")
    cp = pltpu.CompilerParams()
    if "needs_layout_passes" in pltpu.CompilerParams.__dataclass_fields__:
        cp = dataclasses.replace(cp, needs_layout_passes=False)

    @functools.partial(
        pl.kernel,
        compiler_params=cp,
        out_type=(
            jax.ShapeDtypeStruct((NPAD, HW), jnp.int32),
            jax.ShapeDtypeStruct((NPAD, HW), jnp.int32),
        ),
        mesh=mesh,
        scratch_types=(
            [pltpu.VMEM((2 * TPW0 * S,), jnp.int32)]
            + [pltpu.VMEM((C * S, HW), jnp.int32)] * NBUF
            + [pltpu.VMEM((C, HW), jnp.int32)] * NBUF
            + [pltpu.SemaphoreType.DMA] * (2 * NBUF + 2)
        ),
    )
    def sc_kernel(h1_hbm, h2_hbm, i0_hbm, i1_hbm, o0_hbm, o1_hbm,
                  idx_v, *bufs):
        rows = bufs[0:NBUF]
        accs = bufs[NBUF:2 * NBUF]
        gsem = bufs[2 * NBUF:3 * NBUF]
        wsem = bufs[3 * NBUF:4 * NBUF]
        isem = bufs[4 * NBUF:4 * NBUF + 2]
        core = lax.axis_index("c")
        sid = lax.axis_index("s")
        ioff = TPW0 * S

        for ci, tpw in ((0, TPW0), (1, TPW1)):
            if tpw == 0:
                continue
            chunks = tpw // C

            @pl.when(core == ci)
            def _(ci=ci, tpw=tpw, chunks=chunks):
                tbase = sid * tpw + (BASE1 if ci == 1 else 0)
                ibase = tbase * S

                def i_copy(i_hbm, tab, tab_isem):
                    return pltpu.make_async_copy(
                        i_hbm.at[pl.ds(ibase, tpw * S)],
                        idx_v.at[pl.ds(tab * ioff, tpw * S)], tab_isem)

                i_copy(i0_hbm, 0, isem[0]).start()
                i_copy(i1_hbm, 1, isem[1]).start()

                for tab, (t_hbm, i_hbm, o_hbm) in enumerate(
                        ((h1_hbm, i0_hbm, o0_hbm), (h2_hbm, i1_hbm, o1_hbm))):
                    i_copy(i_hbm, tab, isem[tab]).wait()

                    def g_start(cc, b, t_hbm=t_hbm, tab=tab):
                        pltpu.make_async_copy(
                            t_hbm.at[idx_v.at[
                                pl.ds(tab * ioff + cc * (C * S), C * S)]],
                            rows[b], gsem[b]).start()

                    def g_wait(b, t_hbm=t_hbm):
                        pltpu.make_async_copy(
                            t_hbm.at[idx_v.at[pl.ds(0, C * S)]],
                            rows[b], gsem[b]).wait()

                    def w_start(cc, b, o_hbm=o_hbm, tbase=tbase):
                        pltpu.make_async_copy(
                            accs[b], o_hbm.at[pl.ds(tbase + cc * C, C)],
                            wsem[b]).start()

                    def w_wait(b, o_hbm=o_hbm, tbase=tbase):
                        pltpu.make_async_copy(
                            accs[b], o_hbm.at[pl.ds(tbase, C)],
                            wsem[b]).wait()

                    for pre in range(NBUF - 1):
                        g_start(pre, pre)

                    @pl.loop(0, chunks, step=NBUF)
                    def _(c, g_start=g_start, g_wait=g_wait,
                          w_start=w_start, w_wait=w_wait, chunks=chunks):
                        for b in range(NBUF):
                            cc = c + b
                            nxt = cc + NBUF - 1

                            @pl.when(nxt < chunks)
                            def _(nxt=nxt, b=b):
                                g_start(nxt, (b + NBUF - 1) % NBUF)

                            g_wait(b)

                            @pl.when(cc >= NBUF)
                            def _(b=b):
                                w_wait(b)

                            rb, ab = rows[b], accs[b]

                            @pl.loop(0, C)
                            def _(t, rb=rb, ab=ab):
                                r = t * S

                                def bf(x):
                                    return plsc.bitcast(x, jnp.bfloat16)

                                for j in range(HW // 16):
                                    sl = pl.ds(j * 16, 16)
                                    v01 = bf(rb[r, sl]) + bf(rb[r + 1, sl])
                                    v23 = bf(rb[r + 2, sl]) + bf(rb[r + 3, sl])
                                    v45 = bf(rb[r + 4, sl]) + bf(rb[r + 5, sl])
                                    v67 = bf(rb[r + 6, sl]) + bf(rb[r + 7, sl])
                                    ab[t, sl] = plsc.bitcast(
                                        (v01 + v23) + (v45 + v67), jnp.int32)

                            w_start(cc, b)

                    for b in range(NBUF):
                        w_wait(b)

    return sc_kernel(h1p, h2p, idx0, idx1)


def _tc_colsums(s0, s1, wa, wb, fc_b):
    def body(x0_ref, x1_ref, wa_ref, wb_ref, b_ref, out_ref):
        @pl.when(pl.program_id(0) == 0)
        def _():
            out_ref[...] = jnp.zeros_like(out_ref)

        for i, x_ref in enumerate((x0_ref, x1_ref)):
            lo, hi = _unpack_f32(x_ref[...])
            t = jnp.tanh(
                jnp.dot(lo, wa_ref[...], preferred_element_type=jnp.float32)
                + jnp.dot(hi, wb_ref[...], preferred_element_type=jnp.float32)
                + b_ref[...])
            out_ref[i:i + 1, :] += jnp.sum(t, axis=0, keepdims=True)

    return pl.pallas_call(
        body,
        grid=(GRID,),
        in_specs=[
            pl.BlockSpec((BLK, HW), lambda i: (i, 0)),
            pl.BlockSpec((BLK, HW), lambda i: (i, 0)),
            pl.BlockSpec((HW, H), lambda i: (0, 0)),
            pl.BlockSpec((HW, H), lambda i: (0, 0)),
            pl.BlockSpec((1, H), lambda i: (0, 0)),
        ],
        out_specs=pl.BlockSpec((8, H), lambda i: (0, 0)),
        out_shape=jax.ShapeDtypeStruct((8, H), jnp.float32),
    )(s0, s1, wa, wb, fc_b)


def _tc_combine(cs, att, s0, s1, pa, pb, pred_b):
    def body(cs_ref, att_ref, x0_ref, x1_ref, pa_ref, pb_ref, b_ref, out_ref):
        a = att_ref[0, :]
        v0 = jnp.sum(cs_ref[0, :] * a) * (1.0 / N)
        v1 = jnp.sum(cs_ref[1, :] * a) * (1.0 / N)
        m = jnp.maximum(v0, v1)
        e0 = jnp.exp(v0 - m)
        e1 = jnp.exp(v1 - m)
        inv = 1.0 / (e0 + e1)
        b0 = e0 * inv
        b1 = e1 * inv
        lo0, hi0 = _unpack_f32(x0_ref[...])
        lo1, hi1 = _unpack_f32(x1_ref[...])
        z_lo = lo0 * b0 + lo1 * b1
        z_hi = hi0 * b0 + hi1 * b1
        out_ref[...] = jnp.tanh(
            jnp.dot(z_lo, pa_ref[...], preferred_element_type=jnp.float32)
            + jnp.dot(z_hi, pb_ref[...], preferred_element_type=jnp.float32)
            + b_ref[...])

    return pl.pallas_call(
        body,
        grid=(GRID,),
        in_specs=[
            pl.BlockSpec((8, H), lambda i: (0, 0)),
            pl.BlockSpec((1, H), lambda i: (0, 0)),
            pl.BlockSpec((BLK, HW), lambda i: (i, 0)),
            pl.BlockSpec((BLK, HW), lambda i: (i, 0)),
            pl.BlockSpec((HW, H), lambda i: (0, 0)),
            pl.BlockSpec((HW, H), lambda i: (0, 0)),
            pl.BlockSpec((1, H), lambda i: (0, 0)),
        ],
        out_specs=pl.BlockSpec((BLK, H), lambda i: (i, 0)),
        out_shape=jax.ShapeDtypeStruct((N, H), jnp.float32),
    )(cs, att, s0, s1, pa, pb, pred_b)


def kernel(h0, h1, h2, nei_idx0, nei_idx1, fc_W, fc_b, att, pred_W, pred_b):
    del h0
    idx0 = nei_idx0.astype(jnp.int32).reshape(-1)
    idx1 = nei_idx1.astype(jnp.int32).reshape(-1)
    pad = NPAD * S - idx0.shape[0]
    idx0 = jnp.concatenate([idx0, jnp.zeros((pad,), jnp.int32)])
    idx1 = jnp.concatenate([idx1, jnp.zeros((pad,), jnp.int32)])

    cols = jnp.arange(HW)
    ea = jnp.zeros((H, HW), jnp.float32).at[_PERM_LO, cols].set(1.0)
    eb = jnp.zeros((H, HW), jnp.float32).at[_PERM_HI, cols].set(1.0)

    h1p, h2p = _tc_pack(h1, h2, ea, eb)
    s0, s1 = _sc_gather_sums(h1p, h2p, idx0, idx1)

    fc_wt = fc_W.T * (1.0 / S)
    pred_wt = pred_W.T * (1.0 / S)
    cs = _tc_colsums(s0, s1, fc_wt[_PERM_LO, :], fc_wt[_PERM_HI, :],
                     fc_b.reshape(1, H))
    out = _tc_combine(cs, att.reshape(1, H), s0, s1,
                      pred_wt[_PERM_LO, :], pred_wt[_PERM_HI, :],
                      pred_b.reshape(1, H))
    return out
